# baseline (speedup 1.0000x reference)
import math

import jax
import jax.numpy as jnp
from jax import lax
from jax.experimental import pallas as pl
from jax.experimental.pallas import tpu as pltpu
from jax.experimental.pallas import tpu_sc as plsc

B = 1024
HIDDEN = 64
CTX = 4
IN_DIM = HIDDEN * (1 + CTX)
VOCAB = 100000

NC = 2
NS = 16
NW = NC * NS
RPW = B // NW

VT = 2048
NT = math.ceil(VOCAB / VT)


def _sc_gather_body(doc_idx_hbm, words_t_hbm, doc_emb_hbm, voc_emb_hbm,
                    out_hbm, idx_v, rows_v, sem):
    wid = lax.axis_index("s") * NC + lax.axis_index("c")
    base = wid * RPW

    pltpu.sync_copy(doc_idx_hbm.at[pl.ds(base, RPW)], idx_v)
    pltpu.async_copy(doc_emb_hbm.at[idx_v], rows_v, sem).wait()
    pltpu.sync_copy(rows_v, out_hbm.at[pl.ds(base, RPW), pl.ds(0, HIDDEN)])

    for c in range(CTX):
        pltpu.sync_copy(words_t_hbm.at[c, pl.ds(base, RPW)], idx_v)
        pltpu.async_copy(voc_emb_hbm.at[idx_v], rows_v, sem).wait()
        pltpu.sync_copy(
            rows_v,
            out_hbm.at[pl.ds(base, RPW), pl.ds((1 + c) * HIDDEN, HIDDEN)])


def _sc_gather(doc_idx, words_t, doc_emb, voc_emb):
    mesh = plsc.VectorSubcoreMesh(core_axis_name="c", subcore_axis_name="s")
    return pl.kernel(
        _sc_gather_body,
        mesh=mesh,
        out_type=jax.ShapeDtypeStruct((B, IN_DIM), jnp.float32),
        scratch_types=[
            pltpu.VMEM((RPW,), jnp.int32),
            pltpu.VMEM((RPW, HIDDEN), jnp.float32),
            pltpu.SemaphoreType.DMA,
        ],
        compiler_params=pltpu.CompilerParams(use_tc_tiling_on_sc=False),
    )(doc_idx, words_t, doc_emb, voc_emb)


def _tc_body(x_ref, w_ref, b_ref, out_ref, m_ref, s_ref):
    phase = pl.program_id(0)
    j = pl.program_id(1)

    x = x_ref[...].astype(jnp.bfloat16)
    w = w_ref[...].astype(jnp.bfloat16)
    logits = lax.dot_general(
        x, w, (((1,), (1,)), ((), ())),
        preferred_element_type=jnp.float32,
    ) + b_ref[...]

    col = j * VT + lax.broadcasted_iota(jnp.int32, (1, VT), 1)
    valid = col < VOCAB

    @pl.when(phase == 0)
    def _phase0():
        masked = jnp.where(valid, logits, -jnp.inf)
        lm = jnp.max(masked, axis=1, keepdims=True)

        @pl.when(j == 0)
        def _init():
            m_ref[...] = lm
            s_ref[...] = jnp.sum(jnp.exp(masked - lm), axis=1, keepdims=True)

        @pl.when(j > 0)
        def _update():
            m_old = m_ref[...]
            m_new = jnp.maximum(m_old, lm)
            s_ref[...] = s_ref[...] * jnp.exp(m_old - m_new) + jnp.sum(
                jnp.exp(masked - m_new), axis=1, keepdims=True)
            m_ref[...] = m_new

    @pl.when(phase == 1)
    def _phase1():
        out_ref[...] = logits - (m_ref[...] + jnp.log(s_ref[...]))


def _tc_logsoftmax(x, w, b2d):
    return pl.pallas_call(
        _tc_body,
        grid=(2, NT),
        in_specs=[
            pl.BlockSpec((B, IN_DIM), lambda p, j: (0, 0)),
            pl.BlockSpec((VT, IN_DIM), lambda p, j: (j, 0)),
            pl.BlockSpec((1, VT), lambda p, j: (0, j)),
        ],
        out_specs=pl.BlockSpec(
            (B, VT), lambda p, j: (0, jnp.where(p == 0, 0, j))),
        out_shape=jax.ShapeDtypeStruct((B, VOCAB), jnp.float32),
        scratch_shapes=[
            pltpu.VMEM((B, 1), jnp.float32),
            pltpu.VMEM((B, 1), jnp.float32),
        ],
        compiler_params=pltpu.CompilerParams(
            dimension_semantics=("arbitrary", "arbitrary"),
        ),
    )(x, w, b2d)


def kernel(doc_input, words_input, doc_emb, voc_emb, W, b):
    doc_idx = doc_input.reshape(B)
    words_t = words_input.T
    x = _sc_gather(doc_idx, words_t, doc_emb, voc_emb)
    out = _tc_logsoftmax(x, W, b.reshape(1, VOCAB))
    return out.reshape(B, 1, VOCAB)
